# Initial kernel scaffold; baseline (speedup 1.0000x reference)
import jax
import jax.numpy as jnp
from jax.experimental import pallas as pl


def kernel(x_src, x_dst, e, W, b):
    raise NotImplementedError("write your pallas kernel here")


# speedup vs baseline: 2.1922x; 2.1922x over previous
import functools

import jax
import jax.numpy as jnp
from jax import lax
from jax.experimental import pallas as pl
from jax.experimental.pallas import tpu as pltpu
from jax.experimental.pallas import tpu_sc as plsc

N_NODES_K = 10000
N_EDGES_K = 320000
WIDTH_K = 128

NTILES = 32
FPT = WIDTH_K // NTILES
CHUNK = 8000
NCHUNKS = N_EDGES_K // CHUNK
SCANITERS = CHUNK // 16

_GDN = lax.GatherDimensionNumbers(
    offset_dims=(), collapsed_slice_dims=(0,), start_index_map=(0,))


def _vgather(v, idx):
    return lax.gather(v, idx.reshape(16, 1), dimension_numbers=_GDN,
                      slice_sizes=(1,),
                      mode=lax.GatherScatterMode.PROMISE_IN_BOUNDS)


def _seg_min_sc(xs_t, e_src, e_dst):
    mesh = plsc.VectorSubcoreMesh(core_axis_name="c", subcore_axis_name="s")

    @functools.partial(
        pl.kernel,
        mesh=mesh,
        compiler_params=pltpu.CompilerParams(needs_layout_passes=False),
        out_type=jax.ShapeDtypeStruct((NTILES, N_NODES_K * FPT), jnp.float32),
        scratch_types=[
            pltpu.VMEM((CHUNK,), jnp.int32),
            pltpu.VMEM((CHUNK,), jnp.int32),
            pltpu.VMEM((N_NODES_K * FPT,), jnp.float32),
            pltpu.VMEM((N_NODES_K * FPT,), jnp.float32),
        ],
    )
    def seg_min(xs_hbm, esrc_hbm, edst_hbm, out_hbm, srcc, dstc, xsl, acc):
        wid = lax.axis_index("s") * 2 + lax.axis_index("c")
        iota16 = lax.iota(jnp.int32, 16)
        lane4 = iota16 & 3
        rep4 = iota16 >> 2
        base4 = iota16 - lane4
        rot1 = base4 + ((lane4 + 1) & 3)
        rot2 = base4 + ((lane4 + 2) & 3)
        inf16 = jnp.full((16,), jnp.inf, dtype=jnp.float32)

        pltpu.sync_copy(xs_hbm.at[wid], xsl)

        def init_body(i, carry):
            acc[pl.ds(16 * i, 16)] = inf16
            return carry
        lax.fori_loop(0, N_NODES_K * FPT // 16, init_body, jnp.int32(0))

        def chunk_body(c, carry):
            off = c * CHUNK
            pltpu.sync_copy(esrc_hbm.at[pl.ds(off, CHUNK)], srcc)
            pltpu.sync_copy(edst_hbm.at[pl.ds(off, CHUNK)], dstc)

            def vec_body(i, carry2):
                sv = srcc[pl.ds(16 * i, 16)] * FPT
                dv = dstc[pl.ds(16 * i, 16)] * FPT
                dup = ((dv == _vgather(dv, rot1)) |
                       (dv == _vgather(dv, rot2)))
                any_dup = jnp.any(dup)

                @pl.when(jnp.logical_not(any_dup))
                def _fast():
                    for k in range(4):
                        pat = rep4 + (4 * k)
                        six = _vgather(sv, pat) + lane4
                        dix = _vgather(dv, pat) + lane4
                        xv = plsc.load_gather(xsl, [six])
                        av = plsc.load_gather(acc, [dix])
                        plsc.store_scatter(acc, [dix], jnp.minimum(av, xv))

                @pl.when(any_dup)
                def _safe():
                    for e in range(16):
                        pe = jnp.full((16,), e, dtype=jnp.int32)
                        sb = _vgather(sv, pe) + lane4
                        db = _vgather(dv, pe) + lane4
                        xv = plsc.load_gather(xsl, [sb])
                        av = plsc.load_gather(acc, [db])
                        plsc.store_scatter(acc, [db], jnp.minimum(av, xv))

                return carry2

            lax.fori_loop(0, SCANITERS, vec_body, jnp.int32(0))
            return carry

        lax.fori_loop(0, NCHUNKS, chunk_body, jnp.int32(0))

        pltpu.sync_copy(acc, out_hbm.at[wid])

    return seg_min(xs_t, e_src, e_dst)


def _tc_mlp_body(xd_ref, mn_ref, w_ref, b_ref, o_ref):
    xd = xd_ref[...]
    mn = mn_ref[...]
    mx = jnp.where(mn == jnp.inf, 0.0, xd - mn)
    h = (jnp.dot(xd, w_ref[:WIDTH_K, :], precision=lax.Precision.HIGHEST,
                 preferred_element_type=jnp.float32)
         + jnp.dot(mx, w_ref[WIDTH_K:, :], precision=lax.Precision.HIGHEST,
                   preferred_element_type=jnp.float32)
         + b_ref[...])
    o_ref[...] = xd + jnp.where(h > 0, h, 0.01 * h)


def _tc_mlp(x_dst, mins, W, b2):
    blk = 1000
    grid = N_NODES_K // blk
    return pl.pallas_call(
        _tc_mlp_body,
        grid=(grid,),
        in_specs=[
            pl.BlockSpec((blk, WIDTH_K), lambda i: (i, 0)),
            pl.BlockSpec((blk, WIDTH_K), lambda i: (i, 0)),
            pl.BlockSpec((2 * WIDTH_K, WIDTH_K), lambda i: (0, 0)),
            pl.BlockSpec((1, WIDTH_K), lambda i: (0, 0)),
        ],
        out_specs=pl.BlockSpec((blk, WIDTH_K), lambda i: (i, 0)),
        out_shape=jax.ShapeDtypeStruct((N_NODES_K, WIDTH_K), jnp.float32),
    )(x_dst, mins, W, b2)


def kernel(x_src, x_dst, e, W, b):
    xs_t = x_src.reshape(N_NODES_K, NTILES, FPT).transpose(1, 0, 2)
    xs_t = xs_t.reshape(NTILES, N_NODES_K * FPT)
    mins_t = _seg_min_sc(xs_t, e[0], e[1])
    mins = (mins_t.reshape(NTILES, N_NODES_K, FPT)
            .transpose(1, 0, 2).reshape(N_NODES_K, WIDTH_K))
    return _tc_mlp(x_dst, mins, W, b.reshape(1, WIDTH_K))


# speedup vs baseline: 2.3045x; 1.0512x over previous
import functools

import jax
import jax.numpy as jnp
from jax import lax
from jax.experimental import pallas as pl
from jax.experimental.pallas import tpu as pltpu
from jax.experimental.pallas import tpu_sc as plsc

N_NODES_K = 10000
N_EDGES_K = 320000
WIDTH_K = 128

NTILES = 32
FPT = WIDTH_K // NTILES
CHUNK = 2000
NCHUNKS = N_EDGES_K // CHUNK
SCANITERS = CHUNK // 16

_GDN = lax.GatherDimensionNumbers(
    offset_dims=(), collapsed_slice_dims=(0,), start_index_map=(0,))


def _vgather(v, idx):
    return lax.gather(v, idx.reshape(16, 1), dimension_numbers=_GDN,
                      slice_sizes=(1,),
                      mode=lax.GatherScatterMode.PROMISE_IN_BOUNDS)


def _seg_min_sc(xs_t, e_src, e_dst):
    mesh = plsc.VectorSubcoreMesh(core_axis_name="c", subcore_axis_name="s")

    @functools.partial(
        pl.kernel,
        mesh=mesh,
        compiler_params=pltpu.CompilerParams(needs_layout_passes=False),
        out_type=jax.ShapeDtypeStruct((NTILES, N_NODES_K * FPT), jnp.float32),
        scratch_types=[
            [pltpu.VMEM((CHUNK,), jnp.int32) for _ in range(2)],
            [pltpu.VMEM((CHUNK,), jnp.int32) for _ in range(2)],
            pltpu.VMEM((N_NODES_K * FPT,), jnp.float32),
            [pltpu.VMEM((N_NODES_K * FPT,), jnp.float32) for _ in range(2)],
            [pltpu.SemaphoreType.DMA for _ in range(4)],
        ],
    )
    def seg_min(xs_hbm, esrc_hbm, edst_hbm, out_hbm, srccs, dstcs, xsl,
                accs, sems):
        wid = lax.axis_index("s") * 2 + lax.axis_index("c")
        iota16 = lax.iota(jnp.int32, 16)
        lane4 = iota16 & 3
        rep4 = iota16 >> 2
        base4 = iota16 - lane4
        rot1 = base4 + ((lane4 + 1) & 3)
        rot2 = base4 + ((lane4 + 2) & 3)
        inf16 = jnp.full((16,), jnp.inf, dtype=jnp.float32)

        pltpu.sync_copy(xs_hbm.at[wid], xsl)

        def init_body(i, carry):
            accs[0][pl.ds(16 * i, 16)] = inf16
            accs[1][pl.ds(16 * i, 16)] = inf16
            return carry
        lax.fori_loop(0, N_NODES_K * FPT // 16, init_body, jnp.int32(0))

        def fire_chunk(c, p):
            off = jnp.minimum(c, NCHUNKS - 1) * CHUNK
            pltpu.make_async_copy(
                esrc_hbm.at[pl.ds(off, CHUNK)], srccs[p], sems[2 * p]).start()
            pltpu.make_async_copy(
                edst_hbm.at[pl.ds(off, CHUNK)], dstcs[p], sems[2 * p + 1]).start()

        def wait_chunk(p):
            pltpu.make_async_copy(
                esrc_hbm.at[pl.ds(0, CHUNK)], srccs[p], sems[2 * p]).wait()
            pltpu.make_async_copy(
                edst_hbm.at[pl.ds(0, CHUNK)], dstcs[p], sems[2 * p + 1]).wait()

        def process_chunk(p):
            srcc = srccs[p]
            dstc = dstcs[p]

            def vec_body(i, carry2):
                sv = srcc[pl.ds(16 * i, 16)] * FPT
                dv = dstc[pl.ds(16 * i, 16)] * FPT
                dup = ((dv == _vgather(dv, rot1)) |
                       (dv == _vgather(dv, rot2)))
                any_dup = jnp.any(dup)

                @pl.when(jnp.logical_not(any_dup))
                def _fast():
                    for k in range(4):
                        acc = accs[k & 1]
                        pat = rep4 + (4 * k)
                        six = _vgather(sv, pat) + lane4
                        dix = _vgather(dv, pat) + lane4
                        xv = plsc.load_gather(xsl, [six])
                        av = plsc.load_gather(acc, [dix])
                        plsc.store_scatter(acc, [dix], jnp.minimum(av, xv))

                @pl.when(any_dup)
                def _safe():
                    for e in range(16):
                        acc = accs[e & 1]
                        pe = jnp.full((16,), e, dtype=jnp.int32)
                        sb = _vgather(sv, pe) + lane4
                        db = _vgather(dv, pe) + lane4
                        xv = plsc.load_gather(xsl, [sb])
                        av = plsc.load_gather(acc, [db])
                        plsc.store_scatter(acc, [db], jnp.minimum(av, xv))

                return carry2

            lax.fori_loop(0, SCANITERS, vec_body, jnp.int32(0))

        fire_chunk(jnp.int32(0), 0)
        fire_chunk(jnp.int32(1), 1)

        def chunk_body(i, carry):
            c = 2 * i
            wait_chunk(0)
            process_chunk(0)
            fire_chunk(c + 2, 0)
            wait_chunk(1)
            process_chunk(1)
            fire_chunk(c + 3, 1)
            return carry

        lax.fori_loop(0, NCHUNKS // 2, chunk_body, jnp.int32(0))
        wait_chunk(0)
        wait_chunk(1)

        def merge_body(i, carry):
            a = accs[0][pl.ds(16 * i, 16)]
            b = accs[1][pl.ds(16 * i, 16)]
            accs[0][pl.ds(16 * i, 16)] = jnp.minimum(a, b)
            return carry
        lax.fori_loop(0, N_NODES_K * FPT // 16, merge_body, jnp.int32(0))

        pltpu.sync_copy(accs[0], out_hbm.at[wid])

    return seg_min(xs_t, e_src, e_dst)


def _tc_mlp_body(xd_ref, mn_ref, w_ref, b_ref, o_ref):
    xd = xd_ref[...]
    mn = mn_ref[...]
    mx = jnp.where(mn == jnp.inf, 0.0, xd - mn)
    h = (jnp.dot(xd, w_ref[:WIDTH_K, :], precision=lax.Precision.HIGHEST,
                 preferred_element_type=jnp.float32)
         + jnp.dot(mx, w_ref[WIDTH_K:, :], precision=lax.Precision.HIGHEST,
                   preferred_element_type=jnp.float32)
         + b_ref[...])
    o_ref[...] = xd + jnp.where(h > 0, h, 0.01 * h)


def _tc_mlp(x_dst, mins, W, b2):
    blk = 1000
    grid = N_NODES_K // blk
    return pl.pallas_call(
        _tc_mlp_body,
        grid=(grid,),
        in_specs=[
            pl.BlockSpec((blk, WIDTH_K), lambda i: (i, 0)),
            pl.BlockSpec((blk, WIDTH_K), lambda i: (i, 0)),
            pl.BlockSpec((2 * WIDTH_K, WIDTH_K), lambda i: (0, 0)),
            pl.BlockSpec((1, WIDTH_K), lambda i: (0, 0)),
        ],
        out_specs=pl.BlockSpec((blk, WIDTH_K), lambda i: (i, 0)),
        out_shape=jax.ShapeDtypeStruct((N_NODES_K, WIDTH_K), jnp.float32),
    )(x_dst, mins, W, b2)


def kernel(x_src, x_dst, e, W, b):
    xs_t = x_src.reshape(N_NODES_K, NTILES, FPT).transpose(1, 0, 2)
    xs_t = xs_t.reshape(NTILES, N_NODES_K * FPT)
    mins_t = _seg_min_sc(xs_t, e[0], e[1])
    mins = (mins_t.reshape(NTILES, N_NODES_K, FPT)
            .transpose(1, 0, 2).reshape(N_NODES_K, WIDTH_K))
    return _tc_mlp(x_dst, mins, W, b.reshape(1, WIDTH_K))


# speedup vs baseline: 2.4394x; 1.0585x over previous
import functools

import jax
import jax.numpy as jnp
from jax import lax
from jax.experimental import pallas as pl
from jax.experimental.pallas import tpu as pltpu
from jax.experimental.pallas import tpu_sc as plsc

N_NODES_K = 10000
N_EDGES_K = 320000
WIDTH_K = 128

NTILES = 32
FPT = WIDTH_K // NTILES
CHUNK = 2000
NCHUNKS = N_EDGES_K // CHUNK
SCANITERS = CHUNK // 16

_GDN = lax.GatherDimensionNumbers(
    offset_dims=(), collapsed_slice_dims=(0,), start_index_map=(0,))


def _vgather(v, idx):
    return lax.gather(v, idx.reshape(16, 1), dimension_numbers=_GDN,
                      slice_sizes=(1,),
                      mode=lax.GatherScatterMode.PROMISE_IN_BOUNDS)


def _seg_min_sc(xs_t, e_src, e_dst):
    mesh = plsc.VectorSubcoreMesh(core_axis_name="c", subcore_axis_name="s")

    @functools.partial(
        pl.kernel,
        mesh=mesh,
        compiler_params=pltpu.CompilerParams(needs_layout_passes=False),
        out_type=jax.ShapeDtypeStruct((NTILES, N_NODES_K * FPT), jnp.float32),
        scratch_types=[
            [pltpu.VMEM((CHUNK,), jnp.int32) for _ in range(2)],
            [pltpu.VMEM((CHUNK,), jnp.int32) for _ in range(2)],
            pltpu.VMEM((N_NODES_K * FPT,), jnp.float32),
            [pltpu.VMEM((N_NODES_K * FPT,), jnp.float32) for _ in range(2)],
            [pltpu.SemaphoreType.DMA for _ in range(4)],
        ],
    )
    def seg_min(xs_hbm, esrc_hbm, edst_hbm, out_hbm, srccs, dstcs, xsl,
                accs, sems):
        wid = lax.axis_index("s") * 2 + lax.axis_index("c")
        iota16 = lax.iota(jnp.int32, 16)
        lane4 = iota16 & 3
        rep4 = iota16 >> 2
        base4 = iota16 - lane4
        rot1 = base4 + ((lane4 + 1) & 3)
        rot2 = base4 + ((lane4 + 2) & 3)
        inf16 = jnp.full((16,), jnp.inf, dtype=jnp.float32)

        pltpu.sync_copy(xs_hbm.at[wid], xsl)

        def init_body(i, carry):
            accs[0][pl.ds(16 * i, 16)] = inf16
            accs[1][pl.ds(16 * i, 16)] = inf16
            return carry
        lax.fori_loop(0, N_NODES_K * FPT // 16, init_body, jnp.int32(0))

        def fire_chunk(c, p):
            off = jnp.minimum(c, NCHUNKS - 1) * CHUNK
            pltpu.make_async_copy(
                esrc_hbm.at[pl.ds(off, CHUNK)], srccs[p], sems[2 * p]).start()
            pltpu.make_async_copy(
                edst_hbm.at[pl.ds(off, CHUNK)], dstcs[p], sems[2 * p + 1]).start()

        def wait_chunk(p):
            pltpu.make_async_copy(
                esrc_hbm.at[pl.ds(0, CHUNK)], srccs[p], sems[2 * p]).wait()
            pltpu.make_async_copy(
                edst_hbm.at[pl.ds(0, CHUNK)], dstcs[p], sems[2 * p + 1]).wait()

        def process_chunk(p):
            srcc = srccs[p]
            dstc = dstcs[p]

            def prep(i):
                sv = srcc[pl.ds(16 * i, 16)] * FPT
                dv = dstc[pl.ds(16 * i, 16)] * FPT
                dup = ((dv == _vgather(dv, rot1)) |
                       (dv == _vgather(dv, rot2)))
                any_dup = jnp.any(dup)
                ixs = []
                for k in range(4):
                    pat = rep4 + (4 * k)
                    ixs.append(_vgather(sv, pat) + lane4)
                    ixs.append(_vgather(dv, pat) + lane4)
                return (sv, dv, any_dup, *ixs)

            def vec_body(i, carry2):
                sv, dv, any_dup = carry2[0], carry2[1], carry2[2]
                ixs = carry2[3:]
                nxt = prep(jnp.minimum(i + 1, SCANITERS - 1))

                @pl.when(jnp.logical_not(any_dup))
                def _fast():
                    for k in range(4):
                        acc = accs[k & 1]
                        six, dix = ixs[2 * k], ixs[2 * k + 1]
                        xv = plsc.load_gather(xsl, [six])
                        av = plsc.load_gather(acc, [dix])
                        plsc.store_scatter(acc, [dix], jnp.minimum(av, xv))

                @pl.when(any_dup)
                def _safe():
                    for e in range(16):
                        acc = accs[e & 1]
                        pe = jnp.full((16,), e, dtype=jnp.int32)
                        sb = _vgather(sv, pe) + lane4
                        db = _vgather(dv, pe) + lane4
                        xv = plsc.load_gather(xsl, [sb])
                        av = plsc.load_gather(acc, [db])
                        plsc.store_scatter(acc, [db], jnp.minimum(av, xv))

                return nxt

            lax.fori_loop(0, SCANITERS, vec_body, prep(jnp.int32(0)))

        fire_chunk(jnp.int32(0), 0)
        fire_chunk(jnp.int32(1), 1)

        def chunk_body(i, carry):
            c = 2 * i
            wait_chunk(0)
            process_chunk(0)
            fire_chunk(c + 2, 0)
            wait_chunk(1)
            process_chunk(1)
            fire_chunk(c + 3, 1)
            return carry

        lax.fori_loop(0, NCHUNKS // 2, chunk_body, jnp.int32(0))
        wait_chunk(0)
        wait_chunk(1)

        def merge_body(i, carry):
            a = accs[0][pl.ds(16 * i, 16)]
            b = accs[1][pl.ds(16 * i, 16)]
            accs[0][pl.ds(16 * i, 16)] = jnp.minimum(a, b)
            return carry
        lax.fori_loop(0, N_NODES_K * FPT // 16, merge_body, jnp.int32(0))

        pltpu.sync_copy(accs[0], out_hbm.at[wid])

    return seg_min(xs_t, e_src, e_dst)


def _tc_mlp_body(xd_ref, mn_ref, w_ref, b_ref, o_ref):
    xd = xd_ref[...]
    mn = mn_ref[...]
    mx = jnp.where(mn == jnp.inf, 0.0, xd - mn)
    h = (jnp.dot(xd, w_ref[:WIDTH_K, :], precision=lax.Precision.HIGHEST,
                 preferred_element_type=jnp.float32)
         + jnp.dot(mx, w_ref[WIDTH_K:, :], precision=lax.Precision.HIGHEST,
                   preferred_element_type=jnp.float32)
         + b_ref[...])
    o_ref[...] = xd + jnp.where(h > 0, h, 0.01 * h)


def _tc_mlp(x_dst, mins, W, b2):
    blk = 1000
    grid = N_NODES_K // blk
    return pl.pallas_call(
        _tc_mlp_body,
        grid=(grid,),
        in_specs=[
            pl.BlockSpec((blk, WIDTH_K), lambda i: (i, 0)),
            pl.BlockSpec((blk, WIDTH_K), lambda i: (i, 0)),
            pl.BlockSpec((2 * WIDTH_K, WIDTH_K), lambda i: (0, 0)),
            pl.BlockSpec((1, WIDTH_K), lambda i: (0, 0)),
        ],
        out_specs=pl.BlockSpec((blk, WIDTH_K), lambda i: (i, 0)),
        out_shape=jax.ShapeDtypeStruct((N_NODES_K, WIDTH_K), jnp.float32),
    )(x_dst, mins, W, b2)


def kernel(x_src, x_dst, e, W, b):
    xs_t = x_src.reshape(N_NODES_K, NTILES, FPT).transpose(1, 0, 2)
    xs_t = xs_t.reshape(NTILES, N_NODES_K * FPT)
    mins_t = _seg_min_sc(xs_t, e[0], e[1])
    mins = (mins_t.reshape(NTILES, N_NODES_K, FPT)
            .transpose(1, 0, 2).reshape(N_NODES_K, WIDTH_K))
    return _tc_mlp(x_dst, mins, W, b.reshape(1, WIDTH_K))


# speedup vs baseline: 3.2663x; 1.3390x over previous
import functools

import jax
import jax.numpy as jnp
from jax import lax
from jax.experimental import pallas as pl
from jax.experimental.pallas import tpu as pltpu
from jax.experimental.pallas import tpu_sc as plsc

N_NODES_K = 10000
N_EDGES_K = 320000
WIDTH_K = 128

NTILES = 32
FPT = WIDTH_K // NTILES
CHUNK = 2000
NCHUNKS = N_EDGES_K // CHUNK
SCANITERS = CHUNK // 16
BLKV = 5

_GDN = lax.GatherDimensionNumbers(
    offset_dims=(), collapsed_slice_dims=(0,), start_index_map=(0,))


def _vgather(v, idx):
    return lax.gather(v, idx.reshape(16, 1), dimension_numbers=_GDN,
                      slice_sizes=(1,),
                      mode=lax.GatherScatterMode.PROMISE_IN_BOUNDS)


def _seg_min_sc(xs_t, e_src, e_dst):
    mesh = plsc.VectorSubcoreMesh(core_axis_name="c", subcore_axis_name="s")

    @functools.partial(
        pl.kernel,
        mesh=mesh,
        compiler_params=pltpu.CompilerParams(needs_layout_passes=False),
        out_type=jax.ShapeDtypeStruct((NTILES, N_NODES_K * FPT), jnp.float32),
        scratch_types=[
            [pltpu.VMEM((CHUNK,), jnp.int32) for _ in range(2)],
            [pltpu.VMEM((CHUNK,), jnp.int32) for _ in range(2)],
            pltpu.VMEM((N_NODES_K * FPT,), jnp.float32),
            [pltpu.VMEM((N_NODES_K * FPT,), jnp.float32) for _ in range(2)],
            [pltpu.SemaphoreType.DMA for _ in range(4)],
        ],
    )
    def seg_min(xs_hbm, esrc_hbm, edst_hbm, out_hbm, srccs, dstcs, xsl,
                accs, sems):
        wid = lax.axis_index("s") * 2 + lax.axis_index("c")
        iota16 = lax.iota(jnp.int32, 16)
        lane4 = iota16 & 3
        rep4 = iota16 >> 2
        base4 = iota16 - lane4
        rot1 = base4 + ((lane4 + 1) & 3)
        rot2 = base4 + ((lane4 + 2) & 3)
        inf16 = jnp.full((16,), jnp.inf, dtype=jnp.float32)

        pltpu.sync_copy(xs_hbm.at[wid], xsl)

        def init_body(i, carry):
            accs[0][pl.ds(16 * i, 16)] = inf16
            accs[1][pl.ds(16 * i, 16)] = inf16
            return carry
        lax.fori_loop(0, N_NODES_K * FPT // 16, init_body, jnp.int32(0))

        def fire_chunk(c, p):
            off = jnp.minimum(c, NCHUNKS - 1) * CHUNK
            pltpu.make_async_copy(
                esrc_hbm.at[pl.ds(off, CHUNK)], srccs[p], sems[2 * p]).start()
            pltpu.make_async_copy(
                edst_hbm.at[pl.ds(off, CHUNK)], dstcs[p], sems[2 * p + 1]).start()

        def wait_chunk(p):
            pltpu.make_async_copy(
                esrc_hbm.at[pl.ds(0, CHUNK)], srccs[p], sems[2 * p]).wait()
            pltpu.make_async_copy(
                edst_hbm.at[pl.ds(0, CHUNK)], dstcs[p], sems[2 * p + 1]).wait()

        def process_chunk(p):
            srcc = srccs[p]
            dstc = dstcs[p]

            def blk_body(i, carry2):
                b0 = BLKV * i
                dvs = [dstc[pl.ds(16 * (b0 + t), 16)] * FPT
                       for t in range(BLKV)]
                dup = None
                for dv in dvs:
                    d = ((dv == _vgather(dv, rot1)) |
                         (dv == _vgather(dv, rot2)))
                    dup = d if dup is None else (dup | d)
                any_dup = jnp.any(dup)

                @pl.when(jnp.logical_not(any_dup))
                def _fast():
                    for t in range(BLKV):
                        sv = srcc[pl.ds(16 * (b0 + t), 16)] * FPT
                        dv = dvs[t]
                        for k in range(4):
                            acc = accs[k & 1]
                            pat = rep4 + (4 * k)
                            six = _vgather(sv, pat) + lane4
                            dix = _vgather(dv, pat) + lane4
                            xv = plsc.load_gather(xsl, [six])
                            av = plsc.load_gather(acc, [dix])
                            plsc.store_scatter(acc, [dix],
                                               jnp.minimum(av, xv))

                @pl.when(any_dup)
                def _safe():
                    def safe_vec(t, c3):
                        sv = srcc[pl.ds(16 * (b0 + t), 16)] * FPT
                        dv = dstc[pl.ds(16 * (b0 + t), 16)] * FPT
                        for e in range(16):
                            acc = accs[e & 1]
                            pe = jnp.full((16,), e, dtype=jnp.int32)
                            sb = _vgather(sv, pe) + lane4
                            db = _vgather(dv, pe) + lane4
                            xv = plsc.load_gather(xsl, [sb])
                            av = plsc.load_gather(acc, [db])
                            plsc.store_scatter(acc, [db],
                                               jnp.minimum(av, xv))
                        return c3
                    lax.fori_loop(0, BLKV, safe_vec, jnp.int32(0))

                return carry2

            lax.fori_loop(0, SCANITERS // BLKV, blk_body, jnp.int32(0))

        fire_chunk(jnp.int32(0), 0)
        fire_chunk(jnp.int32(1), 1)

        def chunk_body(i, carry):
            c = 2 * i
            wait_chunk(0)
            process_chunk(0)
            fire_chunk(c + 2, 0)
            wait_chunk(1)
            process_chunk(1)
            fire_chunk(c + 3, 1)
            return carry

        lax.fori_loop(0, NCHUNKS // 2, chunk_body, jnp.int32(0))
        wait_chunk(0)
        wait_chunk(1)

        def merge_body(i, carry):
            a = accs[0][pl.ds(16 * i, 16)]
            b = accs[1][pl.ds(16 * i, 16)]
            accs[0][pl.ds(16 * i, 16)] = jnp.minimum(a, b)
            return carry
        lax.fori_loop(0, N_NODES_K * FPT // 16, merge_body, jnp.int32(0))

        pltpu.sync_copy(accs[0], out_hbm.at[wid])

    return seg_min(xs_t, e_src, e_dst)


def _tc_mlp_body(xd_ref, mn_ref, w_ref, b_ref, o_ref):
    xd = xd_ref[...]
    mn = mn_ref[...]
    mx = jnp.where(mn == jnp.inf, 0.0, xd - mn)
    h = (jnp.dot(xd, w_ref[:WIDTH_K, :], precision=lax.Precision.HIGHEST,
                 preferred_element_type=jnp.float32)
         + jnp.dot(mx, w_ref[WIDTH_K:, :], precision=lax.Precision.HIGHEST,
                   preferred_element_type=jnp.float32)
         + b_ref[...])
    o_ref[...] = xd + jnp.where(h > 0, h, 0.01 * h)


def _tc_mlp(x_dst, mins, W, b2):
    blk = 1000
    grid = N_NODES_K // blk
    return pl.pallas_call(
        _tc_mlp_body,
        grid=(grid,),
        in_specs=[
            pl.BlockSpec((blk, WIDTH_K), lambda i: (i, 0)),
            pl.BlockSpec((blk, WIDTH_K), lambda i: (i, 0)),
            pl.BlockSpec((2 * WIDTH_K, WIDTH_K), lambda i: (0, 0)),
            pl.BlockSpec((1, WIDTH_K), lambda i: (0, 0)),
        ],
        out_specs=pl.BlockSpec((blk, WIDTH_K), lambda i: (i, 0)),
        out_shape=jax.ShapeDtypeStruct((N_NODES_K, WIDTH_K), jnp.float32),
    )(x_dst, mins, W, b2)


def kernel(x_src, x_dst, e, W, b):
    xs_t = x_src.reshape(N_NODES_K, NTILES, FPT).transpose(1, 0, 2)
    xs_t = xs_t.reshape(NTILES, N_NODES_K * FPT)
    mins_t = _seg_min_sc(xs_t, e[0], e[1])
    mins = (mins_t.reshape(NTILES, N_NODES_K, FPT)
            .transpose(1, 0, 2).reshape(N_NODES_K, WIDTH_K))
    return _tc_mlp(x_dst, mins, W, b.reshape(1, WIDTH_K))
